# baseline (speedup 1.0000x reference)
import jax
import jax.numpy as jnp
from jax import lax
from jax.experimental import pallas as pl
from jax.experimental.pallas import tpu as pltpu

_B = 128
_NCLF = 64
_NCLS = 10
_HID = _NCLF * _NCLS * 2
_W = 16


def _start(src, dst, sem):
    c = pltpu.make_async_copy(src, dst, sem)
    c.start()
    return c


def _fused_kernel(fcb_h, lpw_h, lpb_h, bb1_h, bw2_h, bb2_h, b1_h, w2_h,
                  b2_h, w3_h, b3_h, sb1_h, sw2_h, sb2_h, env_h, w1_h, sw1_h,
                  logits_ref, lp_ref, stop_ref, slp_ref, sel_ref, clp_ref,
                  bt_ref, fcb_v, lpw_v, lpb_v, bb1_v, bw2_v, bb2_v, b1_v,
                  w2_v, b2_v, w3_v, b3_v, sb1_v, sw2_v, sb2_v, w1rm_v,
                  sw1r_v, sw1m_v, env_v, sems):
    c_fcb = _start(fcb_h, fcb_v, sems.at[0])
    c_lpw = _start(lpw_h, lpw_v, sems.at[1])
    c_lpb = _start(lpb_h, lpb_v, sems.at[2])

    c_fcb.wait(); c_lpw.wait(); c_lpb.wait()
    feat = jnp.maximum(fcb_v[...], 0.0)
    alog = jnp.dot(feat, lpw_v[...],
                   preferred_element_type=jnp.float32) + lpb_v[...]
    m = jnp.max(alog, axis=1, keepdims=True)
    aiota = lax.broadcasted_iota(jnp.int32, alog.shape, 1)
    sel2 = jnp.min(jnp.where(alog == m, aiota, _NCLF), axis=1, keepdims=True)
    sel = sel2[0, 0]
    lse = m + jnp.log(jnp.sum(jnp.exp(alog - m), axis=1, keepdims=True))
    sel_ref[...] = jnp.broadcast_to(sel2, (1, _B))
    clp_ref[...] = jnp.broadcast_to(m - lse, (1, _B))

    base = sel * _NCLS
    a = pl.multiple_of((base // 8) * 8, 8)
    off = base - a
    sa = pl.multiple_of((sel // 8) * 8, 8)
    soff = sel - sa
    c1 = _start(w1_h.at[:, pl.ds(a, _W), :], w1rm_v, sems.at[14])
    c3 = _start(sw1_h.at[pl.ds(a, _W)], sw1r_v, sems.at[16])
    c4 = _start(sw1_h.at[pl.ds(_HID // 2 + a, _W)], sw1m_v, sems.at[17])
    c5 = _start(env_h.at[:, pl.ds(sa, 8), :], env_v, sems.at[18])

    c_bb1 = _start(bb1_h, bb1_v, sems.at[3])
    c_bw2 = _start(bw2_h, bw2_v, sems.at[4])
    c_bb2 = _start(bb2_h, bb2_v, sems.at[5])
    c_b1 = _start(b1_h, b1_v, sems.at[6])
    c_w2 = _start(w2_h, w2_v, sems.at[7])
    c_b2 = _start(b2_h, b2_v, sems.at[8])
    c_w3 = _start(w3_h, w3_v, sems.at[9])
    c_b3 = _start(b3_h, b3_v, sems.at[10])
    c_sb1 = _start(sb1_h, sb1_v, sems.at[11])
    c_sw2 = _start(sw2_h, sw2_v, sems.at[12])
    c_sb2 = _start(sb2_h, sb2_v, sems.at[13])

    c_bb1.wait(); c_bw2.wait(); c_bb2.wait()
    bt = jnp.dot(jnp.maximum(bb1_v[...], 0.0), bw2_v[...],
                 preferred_element_type=jnp.float32) + bb2_v[...]
    bt_ref[...] = jnp.broadcast_to(bt, (1, _B))

    sk = lax.broadcasted_iota(jnp.int32, (_NCLS, _W), 0)
    sj = lax.broadcasted_iota(jnp.int32, (_NCLS, _W), 1)
    S = (sj == sk + off).astype(jnp.float32)
    wi = lax.broadcasted_iota(jnp.int32, (1, _W), 1)
    msk = ((wi >= off) & (wi < off + _NCLS)).astype(jnp.float32)

    c5.wait()
    env8 = env_v[...]
    hot = (lax.broadcasted_iota(jnp.int32, (1, 8, 1), 1) == soff)
    env = jnp.sum(env8 * hot.astype(jnp.float32), axis=1)
    xin = jnp.dot(env, S, preferred_element_type=jnp.float32)

    c1.wait(); c_b1.wait(); c_w2.wait(); c_b2.wait()
    c_w3.wait(); c_b3.wait()
    w1m = jnp.dot(msk, w1rm_v[1], preferred_element_type=jnp.float32)
    x = jnp.dot(xin, w1rm_v[0], preferred_element_type=jnp.float32)
    x = jnp.maximum(x + w1m + b1_v[...], 0.0)
    x = jnp.dot(x, w2_v[...], preferred_element_type=jnp.float32)
    x = jnp.maximum(x + b2_v[...], 0.0)
    logits = jnp.dot(x, w3_v[...],
                     preferred_element_type=jnp.float32) + b3_v[...]
    logits_ref[...] = logits
    lm = jnp.max(logits, axis=1, keepdims=True)
    llse = lm + jnp.log(jnp.sum(jnp.exp(logits - lm), axis=1, keepdims=True))
    lp_ref[...] = logits - llse

    c3.wait(); c4.wait(); c_sb1.wait(); c_sw2.wait(); c_sb2.wait()
    sw1m = jnp.dot(msk, sw1m_v[...], preferred_element_type=jnp.float32)
    f2 = jnp.dot(xin, sw1r_v[...], preferred_element_type=jnp.float32)
    f2 = jnp.maximum(f2 + sw1m + sb1_v[...], 0.0)
    so = jnp.dot(f2, sw2_v[...],
                 preferred_element_type=jnp.float32) + sb2_v[...]
    s0 = so[:, 0:1]
    s1 = so[:, 1:2]
    ii = lax.broadcasted_iota(jnp.int32, (_B, _B), 0)
    jj = lax.broadcasted_iota(jnp.int32, (_B, _B), 1)
    eye = (ii == jj).astype(jnp.float32)
    s0r = jnp.sum(jnp.broadcast_to(s0, (_B, _B)) * eye, axis=0,
                  keepdims=True)
    s1r = jnp.sum(jnp.broadcast_to(s1, (_B, _B)) * eye, axis=0,
                  keepdims=True)
    stop_ref[...] = jnp.where(s0r >= s1r, 0, 1)
    sm = jnp.maximum(s0r, s1r)
    slse = sm + jnp.log(jnp.exp(s0r - sm) + jnp.exp(s1r - sm))
    slp_ref[...] = sm - slse


def kernel(cifar_env_response, act_fc_w, act_fc_b, act_lp_w, act_lp_b,
           base_w1, base_b1, base_w2, base_b2, cls_w1, cls_b1, cls_w2,
           cls_b2, cls_w3, cls_b3, stop_w1, stop_b1, stop_w2, stop_b2):
    del act_fc_w, base_w1
    f32 = jnp.float32
    anym = pl.BlockSpec(memory_space=pl.MemorySpace.ANY)
    outs = pl.pallas_call(
        _fused_kernel,
        in_specs=[anym] * 17,
        out_specs=[pl.BlockSpec(memory_space=pltpu.VMEM)] * 7,
        out_shape=[
            jax.ShapeDtypeStruct((_B, _NCLS), f32),
            jax.ShapeDtypeStruct((_B, _NCLS), f32),
            jax.ShapeDtypeStruct((1, _B), jnp.int32),
            jax.ShapeDtypeStruct((1, _B), f32),
            jax.ShapeDtypeStruct((1, _B), jnp.int32),
            jax.ShapeDtypeStruct((1, _B), f32),
            jax.ShapeDtypeStruct((1, _B), f32),
        ],
        scratch_shapes=[
            pltpu.VMEM((1, 512), f32),
            pltpu.VMEM((512, _NCLF), f32),
            pltpu.VMEM((1, _NCLF), f32),
            pltpu.VMEM((1, 128), f32),
            pltpu.VMEM((128, 1), f32),
            pltpu.VMEM((1, 1), f32),
            pltpu.VMEM((1, 256), f32),
            pltpu.VMEM((256, 256), f32),
            pltpu.VMEM((1, 256), f32),
            pltpu.VMEM((256, _NCLS), f32),
            pltpu.VMEM((1, _NCLS), f32),
            pltpu.VMEM((1, 640), f32),
            pltpu.VMEM((640, 2), f32),
            pltpu.VMEM((1, 2), f32),
            pltpu.VMEM((2, _W, 256), f32),
            pltpu.VMEM((_W, 640), f32),
            pltpu.VMEM((_W, 640), f32),
            pltpu.VMEM((_B, 8, _NCLS), f32),
            pltpu.SemaphoreType.DMA((19,)),
        ],
    )(act_fc_b.reshape(1, -1), act_lp_w, act_lp_b.reshape(1, -1),
      base_b1.reshape(1, -1), base_w2, base_b2.reshape(1, 1),
      cls_b1.reshape(1, -1), cls_w2, cls_b2.reshape(1, -1), cls_w3,
      cls_b3.reshape(1, -1), stop_b1.reshape(1, -1), stop_w2,
      stop_b2.reshape(1, -1), cifar_env_response,
      cls_w1.reshape(2, _HID // 2, 256), stop_w1)
    logits, lp, stop2, slp2, sel2, clp2, bt2 = outs
    return (logits, lp, clp2.reshape(_B), bt2.reshape(_B), slp2.reshape(_B),
            sel2.reshape(_B), stop2.reshape(_B))


# speedup vs baseline: 3.1916x; 3.1916x over previous
import jax
import jax.numpy as jnp
from jax import lax
from jax.experimental import pallas as pl
from jax.experimental.pallas import tpu as pltpu

_B = 128
_NCLF = 64
_NCLS = 10
_HID = _NCLF * _NCLS * 2
_W = 16
_F32 = jnp.float32


def _start(src, dst, sem):
    c = pltpu.make_async_copy(src, dst, sem)
    c.start()
    return c


def _dg(a, b, dims):
    return lax.dot_general(a, b, (dims, ((), ())),
                           preferred_element_type=_F32)


def _fused_kernel(fcb_h, lpwt_h, lpb_h, bb1_h, bw2t_h, bb2_h, b1_h, w2_h,
                  b2_h, w3t_h, b3_h, sb1_h, sw2t_h, sb2_h, envt_h, w1_h,
                  sw1_h, logt_ref, lpt_ref, stop_ref, slp_ref, sel_ref,
                  clp_ref, bt_ref, fcb_v, lpwt_v, lpb_v, bb1_v, bw2t_v,
                  bb2_v, b1_v, w2_v, b2_v, w3t_v, b3_v, sb1_v, sw2t_v,
                  sb2_v, w1rm_v, sw1r_v, sw1m_v, envw_v, sems):
    c_fcb = _start(fcb_h, fcb_v, sems.at[0])
    c_lpw = _start(lpwt_h, lpwt_v, sems.at[1])
    c_lpb = _start(lpb_h, lpb_v, sems.at[2])

    c_fcb.wait(); c_lpw.wait(); c_lpb.wait()
    feat = jnp.maximum(fcb_v[...], 0.0)
    alog = _dg(feat, lpwt_v[...], ((1,), (1,))) + lpb_v[...]
    m = jnp.max(alog, axis=1, keepdims=True)
    aiota = lax.broadcasted_iota(jnp.int32, alog.shape, 1)
    sel2 = jnp.min(jnp.where(alog == m, aiota, _NCLF), axis=1, keepdims=True)
    sel = sel2[0, 0]
    lse = m + jnp.log(jnp.sum(jnp.exp(alog - m), axis=1, keepdims=True))
    sel_ref[...] = jnp.broadcast_to(sel2, (1, _B))
    clp_ref[...] = jnp.broadcast_to(m - lse, (1, _B))

    base = sel * _NCLS
    a = pl.multiple_of((base // 8) * 8, 8)
    off = base - a
    sa = pl.multiple_of((sel // 8) * 8, 8)
    soff = sel - sa
    c1 = _start(w1_h.at[:, pl.ds(a, _W), :], w1rm_v, sems.at[14])
    c3 = _start(sw1_h.at[pl.ds(a, _W)], sw1r_v, sems.at[15])
    c4 = _start(sw1_h.at[pl.ds(_HID // 2 + a, _W)], sw1m_v, sems.at[16])
    c5 = _start(envt_h.at[:, pl.ds(sa, 8), :], envw_v, sems.at[17])

    c_bb1 = _start(bb1_h, bb1_v, sems.at[3])
    c_bw2 = _start(bw2t_h, bw2t_v, sems.at[4])
    c_bb2 = _start(bb2_h, bb2_v, sems.at[5])
    c_b1 = _start(b1_h, b1_v, sems.at[6])
    c_w2 = _start(w2_h, w2_v, sems.at[7])
    c_b2 = _start(b2_h, b2_v, sems.at[8])
    c_w3 = _start(w3t_h, w3t_v, sems.at[9])
    c_b3 = _start(b3_h, b3_v, sems.at[10])
    c_sb1 = _start(sb1_h, sb1_v, sems.at[11])
    c_sw2 = _start(sw2t_h, sw2t_v, sems.at[12])
    c_sb2 = _start(sb2_h, sb2_v, sems.at[13])

    c_bb1.wait(); c_bw2.wait(); c_bb2.wait()
    bt = jnp.sum(jnp.maximum(bb1_v[...], 0.0) * bw2t_v[...],
                 axis=1, keepdims=True) + bb2_v[...]
    bt_ref[...] = jnp.broadcast_to(bt, (1, _B))

    sk = lax.broadcasted_iota(jnp.int32, (_NCLS, _W), 0)
    sj = lax.broadcasted_iota(jnp.int32, (_NCLS, _W), 1)
    S = (sj == sk + off).astype(_F32)
    wi = lax.broadcasted_iota(jnp.int32, (_W, _B), 0)
    mskcol = ((wi >= off) & (wi < off + _NCLS)).astype(_F32)
    ones_r = jnp.ones((1, _B), _F32)

    c5.wait()
    envw = envw_v[...]
    hot = (lax.broadcasted_iota(jnp.int32, (1, 8, 1), 1) == soff)
    envt = jnp.sum(envw * hot.astype(_F32), axis=1)
    xint = _dg(S, envt, ((0,), (0,)))

    c1.wait(); c_b1.wait(); c_w2.wait(); c_b2.wait()
    c_w3.wait(); c_b3.wait()
    x = _dg(w1rm_v[0], xint, ((0,), (0,)))
    x = x + _dg(w1rm_v[1], mskcol, ((0,), (0,)))
    x = jnp.maximum(x + _dg(b1_v[...], ones_r, ((0,), (0,))), 0.0)
    x = _dg(w2_v[...], x, ((0,), (0,)))
    x = jnp.maximum(x + _dg(b2_v[...], ones_r, ((0,), (0,))), 0.0)
    logt = _dg(w3t_v[...], x, ((1,), (0,)))
    logt = logt + _dg(b3_v[...], ones_r, ((0,), (0,)))
    logt_ref[...] = logt
    lm = jnp.max(logt, axis=0, keepdims=True)
    llse = lm + jnp.log(jnp.sum(jnp.exp(logt - lm), axis=0, keepdims=True))
    lpt_ref[...] = logt - llse

    c3.wait(); c4.wait(); c_sb1.wait(); c_sw2.wait(); c_sb2.wait()
    f2 = _dg(sw1r_v[...], xint, ((0,), (0,)))
    f2 = f2 + _dg(sw1m_v[...], mskcol, ((0,), (0,)))
    f2 = jnp.maximum(f2 + _dg(sb1_v[...], ones_r, ((0,), (0,))), 0.0)
    sot = _dg(sw2t_v[...], f2, ((1,), (0,)))
    sot = sot + _dg(sb2_v[...], ones_r, ((0,), (0,)))
    s0r = sot[0:1, :]
    s1r = sot[1:2, :]
    stop_ref[...] = jnp.where(s0r >= s1r, 0, 1)
    sm = jnp.maximum(s0r, s1r)
    slse = sm + jnp.log(jnp.exp(s0r - sm) + jnp.exp(s1r - sm))
    slp_ref[...] = sm - slse


def kernel(cifar_env_response, act_fc_w, act_fc_b, act_lp_w, act_lp_b,
           base_w1, base_b1, base_w2, base_b2, cls_w1, cls_b1, cls_w2,
           cls_b2, cls_w3, cls_b3, stop_w1, stop_b1, stop_w2, stop_b2):
    del act_fc_w, base_w1
    anym = pl.BlockSpec(memory_space=pl.MemorySpace.ANY)
    outs = pl.pallas_call(
        _fused_kernel,
        in_specs=[anym] * 17,
        out_specs=[pl.BlockSpec(memory_space=pltpu.VMEM)] * 7,
        out_shape=[
            jax.ShapeDtypeStruct((_NCLS, _B), _F32),
            jax.ShapeDtypeStruct((_NCLS, _B), _F32),
            jax.ShapeDtypeStruct((1, _B), jnp.int32),
            jax.ShapeDtypeStruct((1, _B), _F32),
            jax.ShapeDtypeStruct((1, _B), jnp.int32),
            jax.ShapeDtypeStruct((1, _B), _F32),
            jax.ShapeDtypeStruct((1, _B), _F32),
        ],
        scratch_shapes=[
            pltpu.VMEM((1, 512), _F32),
            pltpu.VMEM((_NCLF, 512), _F32),
            pltpu.VMEM((1, _NCLF), _F32),
            pltpu.VMEM((1, 128), _F32),
            pltpu.VMEM((1, 128), _F32),
            pltpu.VMEM((1, 1), _F32),
            pltpu.VMEM((1, 256), _F32),
            pltpu.VMEM((256, 256), _F32),
            pltpu.VMEM((1, 256), _F32),
            pltpu.VMEM((_NCLS, 256), _F32),
            pltpu.VMEM((1, _NCLS), _F32),
            pltpu.VMEM((1, 640), _F32),
            pltpu.VMEM((2, 640), _F32),
            pltpu.VMEM((1, 2), _F32),
            pltpu.VMEM((2, _W, 256), _F32),
            pltpu.VMEM((_W, 640), _F32),
            pltpu.VMEM((_W, 640), _F32),
            pltpu.VMEM((_NCLS, 8, _B), _F32),
            pltpu.SemaphoreType.DMA((18,)),
        ],
    )(act_fc_b.reshape(1, -1), act_lp_w.T, act_lp_b.reshape(1, -1),
      base_b1.reshape(1, -1), base_w2.T, base_b2.reshape(1, 1),
      cls_b1.reshape(1, -1), cls_w2, cls_b2.reshape(1, -1), cls_w3.T,
      cls_b3.reshape(1, -1), stop_b1.reshape(1, -1), stop_w2.T,
      stop_b2.reshape(1, -1), jnp.transpose(cifar_env_response, (2, 1, 0)),
      cls_w1.reshape(2, _HID // 2, 256), stop_w1)
    logt, lpt, stop2, slp2, sel2, clp2, bt2 = outs
    return (logt.T, lpt.T, clp2.reshape(_B), bt2.reshape(_B),
            slp2.reshape(_B), sel2.reshape(_B), stop2.reshape(_B))
